# baseline (speedup 1.0000x reference)
import functools

import jax
import jax.numpy as jnp
from jax import lax
from jax.experimental import pallas as pl
from jax.experimental.pallas import tpu as pltpu
from jax.experimental.pallas import tpu_sc as plsc

_E = 6_400_000
_NC = 2
_NS = 16
_NW = _NC * _NS
_PER_W = _E // _NW
_CHUNK = 20_000
_NCHUNK = _PER_W // _CHUNK
_L = 16


def _body(types_hbm, table_hbm, out_hbm, table_v, exp_v,
          idx0, idx1, out0, out1, sin0, sin1, sout0, sout1):
    wid = lax.axis_index("s") * _NC + lax.axis_index("c")
    base = wid * _PER_W

    pltpu.sync_copy(table_hbm, table_v)
    exp_v[pl.ds(0, _L)] = jnp.exp(table_v[pl.ds(0, _L)])
    exp_v[pl.ds(_L, _L)] = jnp.exp(table_v[pl.ds(_L, _L)])

    idx_bufs = (idx0, idx1)
    out_bufs = (out0, out1)
    sin = (sin0, sin1)
    sout = (sout0, sout1)

    in_copies = [None] * _NCHUNK
    out_copies = [None] * _NCHUNK
    in_copies[0] = pltpu.async_copy(
        types_hbm.at[pl.ds(base, _CHUNK)], idx_bufs[0], sin[0])

    for ci in range(_NCHUNK):
        b = ci % 2
        if ci + 1 < _NCHUNK:
            nb = (ci + 1) % 2
            in_copies[ci + 1] = pltpu.async_copy(
                types_hbm.at[pl.ds(base + (ci + 1) * _CHUNK, _CHUNK)],
                idx_bufs[nb], sin[nb])
        in_copies[ci].wait()
        if ci >= 2:
            out_copies[ci - 2].wait()
        iv = idx_bufs[b]
        ov = out_bufs[b]

        @plsc.parallel_loop(0, _CHUNK, step=_L, unroll=16)
        def _gather(k, iv=iv, ov=ov):
            ov[pl.ds(k, _L)] = plsc.load_gather(exp_v, [iv[pl.ds(k, _L)]])
        out_copies[ci] = pltpu.async_copy(
            ov, out_hbm.at[pl.ds(base + ci * _CHUNK, _CHUNK)], sout[b])

    out_copies[_NCHUNK - 2].wait()
    out_copies[_NCHUNK - 1].wait()


_sc_call = functools.partial(
    pl.kernel,
    out_type=jax.ShapeDtypeStruct((_E,), jnp.float32),
    mesh=plsc.VectorSubcoreMesh(
        core_axis_name="c", subcore_axis_name="s", num_cores=_NC, num_subcores=_NS
    ),
    compiler_params=pltpu.CompilerParams(needs_layout_passes=False),
    scratch_types=[
        pltpu.VMEM((32,), jnp.float32),
        pltpu.VMEM((32,), jnp.float32),
        pltpu.VMEM((_CHUNK,), jnp.int32),
        pltpu.VMEM((_CHUNK,), jnp.int32),
        pltpu.VMEM((_CHUNK,), jnp.float32),
        pltpu.VMEM((_CHUNK,), jnp.float32),
        pltpu.SemaphoreType.DMA,
        pltpu.SemaphoreType.DMA,
        pltpu.SemaphoreType.DMA,
        pltpu.SemaphoreType.DMA,
    ],
)(_body)


def kernel(edge_types, importance):
    table = jnp.pad(importance.reshape(-1), (0, 32 - importance.shape[0]))
    return _sc_call(edge_types.astype(jnp.int32), table)


# speedup vs baseline: 1.0642x; 1.0642x over previous
import functools

import jax
import jax.numpy as jnp
from jax import lax
from jax.experimental import pallas as pl
from jax.experimental.pallas import tpu as pltpu
from jax.experimental.pallas import tpu_sc as plsc

_E = 6_400_000
_NC = 2
_NS = 16
_NW = _NC * _NS
_PER_W = _E // _NW
_CHUNK = 20_000
_NCHUNK = _PER_W // _CHUNK
_L = 16


def _body(types_hbm, table_hbm, out_hbm, table_v, exp_v,
          idx0, idx1, out0, out1, sin0, sin1, sout0, sout1):
    wid = lax.axis_index("s") * _NC + lax.axis_index("c")
    base = wid * _PER_W

    pltpu.sync_copy(table_hbm, table_v.at[pl.ds(0, 17)])
    exp_v[pl.ds(0, _L)] = jnp.exp(table_v[pl.ds(0, _L)])
    exp_v[pl.ds(_L, _L)] = jnp.exp(table_v[pl.ds(_L, _L)])

    idx_bufs = (idx0, idx1)
    out_bufs = (out0, out1)
    sin = (sin0, sin1)
    sout = (sout0, sout1)

    for b in (0, 1):
        pltpu.async_copy(
            types_hbm.at[pl.ds(base + b * _CHUNK, _CHUNK)], idx_bufs[b], sin[b])

    def pair_body(g, carry):
        for b in (0, 1):
            ci = g * 2 + b
            off = base + ci * _CHUNK
            iv = idx_bufs[b]
            ov = out_bufs[b]
            pltpu.make_async_copy(
                types_hbm.at[pl.ds(off, _CHUNK)], iv, sin[b]).wait()

            @pl.when(ci >= 2)
            def _():
                pltpu.make_async_copy(
                    ov, out_hbm.at[pl.ds(off, _CHUNK)], sout[b]).wait()

            @plsc.parallel_loop(0, _CHUNK, step=_L, unroll=8)
            def _gather(k, iv=iv, ov=ov):
                ov[pl.ds(k, _L)] = plsc.load_gather(exp_v, [iv[pl.ds(k, _L)]])

            pltpu.async_copy(ov, out_hbm.at[pl.ds(off, _CHUNK)], sout[b])

            @pl.when(ci + 2 < _NCHUNK)
            def _():
                pltpu.async_copy(
                    types_hbm.at[pl.ds(off + 2 * _CHUNK, _CHUNK)], iv, sin[b])
        return carry

    lax.fori_loop(0, _NCHUNK // 2, pair_body, 0)
    for b in (0, 1):
        pltpu.make_async_copy(
            out_bufs[b], out_hbm.at[pl.ds(base, _CHUNK)], sout[b]).wait()


_sc_call = functools.partial(
    pl.kernel,
    out_type=jax.ShapeDtypeStruct((_E,), jnp.float32),
    mesh=plsc.VectorSubcoreMesh(
        core_axis_name="c", subcore_axis_name="s", num_cores=_NC, num_subcores=_NS
    ),
    compiler_params=pltpu.CompilerParams(needs_layout_passes=False),
    scratch_types=[
        pltpu.VMEM((32,), jnp.float32),
        pltpu.VMEM((32,), jnp.float32),
        pltpu.VMEM((_CHUNK,), jnp.int32),
        pltpu.VMEM((_CHUNK,), jnp.int32),
        pltpu.VMEM((_CHUNK,), jnp.float32),
        pltpu.VMEM((_CHUNK,), jnp.float32),
        pltpu.SemaphoreType.DMA,
        pltpu.SemaphoreType.DMA,
        pltpu.SemaphoreType.DMA,
        pltpu.SemaphoreType.DMA,
    ],
)(_body)


def kernel(edge_types, importance):
    return _sc_call(edge_types.astype(jnp.int32), importance.reshape(-1))


# speedup vs baseline: 1.0660x; 1.0017x over previous
import functools

import jax
import jax.numpy as jnp
from jax import lax
from jax.experimental import pallas as pl
from jax.experimental.pallas import tpu as pltpu
from jax.experimental.pallas import tpu_sc as plsc

_E = 6_400_000
_NC = 2
_NS = 16
_NW = _NC * _NS
_PER_W = _E // _NW
_CHUNK = 20_000
_NCHUNK = _PER_W // _CHUNK
_L = 16


def _body(types_hbm, table_hbm, out_hbm, table_v, exp_v,
          idx0, idx1, out0, out1, sin0, sin1, sout0, sout1):
    wid = lax.axis_index("s") * _NC + lax.axis_index("c")
    base = wid * _PER_W

    pltpu.sync_copy(table_hbm, table_v.at[pl.ds(0, 17)])
    exp_v[pl.ds(0, _L)] = jnp.exp(table_v[pl.ds(0, _L)])
    exp_v[pl.ds(_L, _L)] = jnp.exp(table_v[pl.ds(_L, _L)])

    idx_bufs = (idx0, idx1)
    out_bufs = (out0, out1)
    sin = (sin0, sin1)
    sout = (sout0, sout1)

    for b in (0, 1):
        pltpu.async_copy(
            types_hbm.at[pl.ds(base + b * _CHUNK, _CHUNK)], idx_bufs[b], sin[b])

    def pair_body(g, carry):
        for b in (0, 1):
            ci = g * 2 + b
            off = base + ci * _CHUNK
            iv = idx_bufs[b]
            ov = out_bufs[b]
            pltpu.make_async_copy(
                types_hbm.at[pl.ds(off, _CHUNK)], iv, sin[b]).wait()

            @pl.when(ci >= 2)
            def _():
                pltpu.make_async_copy(
                    ov, out_hbm.at[pl.ds(off, _CHUNK)], sout[b]).wait()

            @plsc.parallel_loop(0, _CHUNK, step=_L, unroll=8)
            def _gather(k, iv=iv, ov=ov):
                ov[pl.ds(k, _L)] = plsc.load_gather(exp_v, [iv[pl.ds(k, _L)]])

            pltpu.async_copy(ov, out_hbm.at[pl.ds(off, _CHUNK)], sout[b])

            @pl.when(ci + 2 < _NCHUNK)
            def _():
                pltpu.async_copy(
                    types_hbm.at[pl.ds(off + 2 * _CHUNK, _CHUNK)], iv, sin[b])
        return carry

    lax.fori_loop(0, _NCHUNK // 2, pair_body, 0)
    for b in (0, 1):
        pltpu.make_async_copy(
            out_bufs[b], out_hbm.at[pl.ds(base, _CHUNK)], sout[b]).wait()


_sc_call = functools.partial(
    pl.kernel,
    out_type=jax.ShapeDtypeStruct((_E,), jnp.float32),
    mesh=plsc.VectorSubcoreMesh(
        core_axis_name="c", subcore_axis_name="s", num_cores=_NC, num_subcores=_NS
    ),
    compiler_params=pltpu.CompilerParams(
        needs_layout_passes=False, skip_device_barrier=True
    ),
    scratch_types=[
        pltpu.VMEM((32,), jnp.float32),
        pltpu.VMEM((32,), jnp.float32),
        pltpu.VMEM((_CHUNK,), jnp.int32),
        pltpu.VMEM((_CHUNK,), jnp.int32),
        pltpu.VMEM((_CHUNK,), jnp.float32),
        pltpu.VMEM((_CHUNK,), jnp.float32),
        pltpu.SemaphoreType.DMA,
        pltpu.SemaphoreType.DMA,
        pltpu.SemaphoreType.DMA,
        pltpu.SemaphoreType.DMA,
    ],
)(_body)


def kernel(edge_types, importance):
    return _sc_call(edge_types.astype(jnp.int32), importance.reshape(-1))


# speedup vs baseline: 1.1046x; 1.0362x over previous
import functools

import jax
import jax.numpy as jnp
from jax import lax
from jax.experimental import pallas as pl
from jax.experimental.pallas import tpu as pltpu
from jax.experimental.pallas import tpu_sc as plsc

_E = 6_400_000
_NC = 2
_NS = 16
_NW = _NC * _NS
_PER_W = _E // _NW
_NBUF = 4
_CHUNK = 10_000
_NCHUNK = _PER_W // _CHUNK
_L = 16


def _body(types_hbm, table_hbm, out_hbm, table_v, exp_v, *bufs):
    idx_bufs = bufs[0:_NBUF]
    out_bufs = bufs[_NBUF:2 * _NBUF]
    sin = bufs[2 * _NBUF:3 * _NBUF]
    sout = bufs[3 * _NBUF:4 * _NBUF]

    wid = lax.axis_index("s") * _NC + lax.axis_index("c")
    base = wid * _PER_W

    pltpu.sync_copy(table_hbm, table_v.at[pl.ds(0, 17)])
    exp_v[pl.ds(0, _L)] = jnp.exp(table_v[pl.ds(0, _L)])
    exp_v[pl.ds(_L, _L)] = jnp.exp(table_v[pl.ds(_L, _L)])

    for b in range(_NBUF):
        pltpu.async_copy(
            types_hbm.at[pl.ds(base + b * _CHUNK, _CHUNK)], idx_bufs[b], sin[b])

    def group_body(g, carry):
        for b in range(_NBUF):
            ci = g * _NBUF + b
            off = base + ci * _CHUNK
            iv = idx_bufs[b]
            ov = out_bufs[b]
            pltpu.make_async_copy(
                types_hbm.at[pl.ds(off, _CHUNK)], iv, sin[b]).wait()

            @pl.when(ci >= _NBUF)
            def _():
                pltpu.make_async_copy(
                    ov, out_hbm.at[pl.ds(off, _CHUNK)], sout[b]).wait()

            @plsc.parallel_loop(0, _CHUNK, step=_L, unroll=8)
            def _gather(k, iv=iv, ov=ov):
                ov[pl.ds(k, _L)] = plsc.load_gather(exp_v, [iv[pl.ds(k, _L)]])

            pltpu.async_copy(ov, out_hbm.at[pl.ds(off, _CHUNK)], sout[b])

            @pl.when(ci + _NBUF < _NCHUNK)
            def _():
                pltpu.async_copy(
                    types_hbm.at[pl.ds(off + _NBUF * _CHUNK, _CHUNK)], iv, sin[b])
        return carry

    lax.fori_loop(0, _NCHUNK // _NBUF, group_body, 0)
    for b in range(_NBUF):
        pltpu.make_async_copy(
            out_bufs[b], out_hbm.at[pl.ds(base, _CHUNK)], sout[b]).wait()


_sc_call = functools.partial(
    pl.kernel,
    out_type=jax.ShapeDtypeStruct((_E,), jnp.float32),
    mesh=plsc.VectorSubcoreMesh(
        core_axis_name="c", subcore_axis_name="s", num_cores=_NC, num_subcores=_NS
    ),
    compiler_params=pltpu.CompilerParams(
        needs_layout_passes=False, skip_device_barrier=True
    ),
    scratch_types=(
        [
            pltpu.VMEM((32,), jnp.float32),
            pltpu.VMEM((32,), jnp.float32),
        ]
        + [pltpu.VMEM((_CHUNK,), jnp.int32) for _ in range(_NBUF)]
        + [pltpu.VMEM((_CHUNK,), jnp.float32) for _ in range(_NBUF)]
        + [pltpu.SemaphoreType.DMA for _ in range(2 * _NBUF)]
    ),
)(_body)


def kernel(edge_types, importance):
    return _sc_call(edge_types.astype(jnp.int32), importance.reshape(-1))


# speedup vs baseline: 1.1109x; 1.0058x over previous
import functools

import jax
import jax.numpy as jnp
from jax import lax
from jax.experimental import pallas as pl
from jax.experimental.pallas import tpu as pltpu
from jax.experimental.pallas import tpu_sc as plsc

_E = 6_400_000
_NC = 2
_NS = 16
_NW = _NC * _NS
_PER_W = _E // _NW
_NBUF = 5
_CHUNK = 8_000
_NCHUNK = _PER_W // _CHUNK
_L = 16


def _body(types_hbm, table_hbm, out_hbm, table_v, exp_v, *bufs):
    idx_bufs = bufs[0:_NBUF]
    out_bufs = bufs[_NBUF:2 * _NBUF]
    sin = bufs[2 * _NBUF:3 * _NBUF]
    sout = bufs[3 * _NBUF:4 * _NBUF]

    wid = lax.axis_index("s") * _NC + lax.axis_index("c")
    base = wid * _PER_W

    pltpu.sync_copy(table_hbm, table_v.at[pl.ds(0, 17)])
    exp_v[pl.ds(0, _L)] = jnp.exp(table_v[pl.ds(0, _L)])
    exp_v[pl.ds(_L, _L)] = jnp.exp(table_v[pl.ds(_L, _L)])

    for b in range(_NBUF):
        pltpu.async_copy(
            types_hbm.at[pl.ds(base + b * _CHUNK, _CHUNK)], idx_bufs[b], sin[b])

    def group_body(g, carry):
        for b in range(_NBUF):
            ci = g * _NBUF + b
            off = base + ci * _CHUNK
            iv = idx_bufs[b]
            ov = out_bufs[b]
            pltpu.make_async_copy(
                types_hbm.at[pl.ds(off, _CHUNK)], iv, sin[b]).wait()

            @pl.when(ci >= _NBUF)
            def _():
                pltpu.make_async_copy(
                    ov, out_hbm.at[pl.ds(off, _CHUNK)], sout[b]).wait()

            @plsc.parallel_loop(0, _CHUNK, step=_L, unroll=8)
            def _gather(k, iv=iv, ov=ov):
                ov[pl.ds(k, _L)] = plsc.load_gather(exp_v, [iv[pl.ds(k, _L)]])

            pltpu.async_copy(ov, out_hbm.at[pl.ds(off, _CHUNK)], sout[b])

            @pl.when(ci + _NBUF < _NCHUNK)
            def _():
                pltpu.async_copy(
                    types_hbm.at[pl.ds(off + _NBUF * _CHUNK, _CHUNK)], iv, sin[b])
        return carry

    lax.fori_loop(0, _NCHUNK // _NBUF, group_body, 0)
    for b in range(_NBUF):
        pltpu.make_async_copy(
            out_bufs[b], out_hbm.at[pl.ds(base, _CHUNK)], sout[b]).wait()


_sc_call = functools.partial(
    pl.kernel,
    out_type=jax.ShapeDtypeStruct((_E,), jnp.float32),
    mesh=plsc.VectorSubcoreMesh(
        core_axis_name="c", subcore_axis_name="s", num_cores=_NC, num_subcores=_NS
    ),
    compiler_params=pltpu.CompilerParams(
        needs_layout_passes=False, skip_device_barrier=True
    ),
    scratch_types=(
        [
            pltpu.VMEM((32,), jnp.float32),
            pltpu.VMEM((32,), jnp.float32),
        ]
        + [pltpu.VMEM((_CHUNK,), jnp.int32) for _ in range(_NBUF)]
        + [pltpu.VMEM((_CHUNK,), jnp.float32) for _ in range(_NBUF)]
        + [pltpu.SemaphoreType.DMA for _ in range(2 * _NBUF)]
    ),
)(_body)


def kernel(edge_types, importance):
    return _sc_call(edge_types.astype(jnp.int32), importance.reshape(-1))
